# Initial kernel scaffold; baseline (speedup 1.0000x reference)
import jax
import jax.numpy as jnp
from jax.experimental import pallas as pl


def kernel(x, w0, b0, scale0, shift0, w1, b1, scale1, shift1, w2, b2, scale2, shift2, w3, b3, scale3, shift3):
    raise NotImplementedError("write your pallas kernel here")


# speedup vs baseline: 35.9107x; 35.9107x over previous
import functools

import jax
import jax.numpy as jnp
from jax.experimental import pallas as pl
from jax.experimental.pallas import tpu as pltpu

_TB = 64


def _phases(y, hp):
    u, v, tb, c = y.shape
    yp = jnp.pad(y, ((1, hp - u - 1), (1, hp - v - 1), (0, 0), (0, 0)))
    ph = []
    for a in range(2):
        ya = yp.reshape(hp // 2, 2, hp, tb, c)[:, a]
        ph.append([ya.reshape(hp // 2, hp // 2, 2, tb, c)[:, :, b]
                   for b in range(2)])
    return ph


def _conv5s2(y, w_ref, s_ref, t_ref, hp, wo, relu):
    u, v, tb, cin = y.shape
    ph = _phases(y, hp)
    acc = None
    for kh in range(5):
        for kw in range(5):
            tap = ph[kh % 2][kw % 2][kh // 2:kh // 2 + wo,
                                     kw // 2:kw // 2 + wo]
            lhs = tap.reshape(wo * wo * tb, cin)
            d = jnp.dot(lhs, w_ref[kh, kw],
                        preferred_element_type=jnp.float32)
            acc = d if acc is None else acc + d
    out = acc * s_ref[...] + t_ref[...]
    if relu:
        out = jnp.maximum(out, 0.0)
    cout = out.shape[-1]
    return out.astype(jnp.bfloat16).reshape(wo, wo, tb, cout)


def _fused_body(p1_ref, w1_ref, s1_ref, t1_ref, w2_ref, s2_ref, t2_ref,
                w3_ref, s3_ref, t3_ref, w4_ref, s4_ref, t4_ref,
                z_ref, rf_ref, *, tb):
    p1 = p1_ref[...].reshape(15 * 15 * tb, 75)
    acc = jnp.dot(p1, w1_ref[...], preferred_element_type=jnp.float32)
    y = jnp.maximum(acc * s1_ref[...] + t1_ref[...], 0.0)
    y = y.astype(jnp.bfloat16).reshape(15, 15, tb, 32)

    y = _conv5s2(y, w2_ref, s2_ref, t2_ref, 18, 7, True)
    y = _conv5s2(y, w3_ref, s3_ref, t3_ref, 10, 3, True)

    yp = jnp.pad(y, ((1, 1), (1, 1), (0, 0), (0, 0)))
    acc = None
    for kh in range(5):
        for kw in range(5):
            d = jnp.dot(yp[kh, kw], w4_ref[kh, kw],
                        preferred_element_type=jnp.float32)
            acc = d if acc is None else acc + d
    yout = acc * s4_ref[...] + t4_ref[...]

    z_ref[...] = yout[:, :100]
    rf_ref[...] = 1.0 / (1.0 + jnp.exp(-yout[:, 100:101]))


def kernel(x, w0, b0, scale0, shift0, w1, b1, scale1, shift1,
           w2, b2, scale2, shift2, w3, b3, scale3, shift3):
    B = x.shape[0]
    tb = _TB

    xh = jnp.transpose(x, (2, 3, 0, 1)).astype(jnp.bfloat16)
    xp = jnp.pad(xh, ((1, 1), (1, 1), (0, 0), (0, 0)))
    cols = [xp[kh:kh + 29:2, kw:kw + 29:2]
            for kh in range(5) for kw in range(5)]
    p1 = jnp.concatenate(cols, axis=-1)

    params = []
    for w, b, s, t in zip((w0, w1, w2, w3), (b0, b1, b2, b3),
                          (scale0, scale1, scale2, scale3),
                          (shift0, shift1, shift2, shift3)):
        cout = w.shape[-1]
        params.append((w.astype(jnp.bfloat16),
                       s.reshape(1, cout).astype(jnp.float32),
                       (b * s + t).reshape(1, cout).astype(jnp.float32)))
    w1m = params[0][0].reshape(75, 32)

    full = lambda shape: pl.BlockSpec(shape, lambda i: (0,) * len(shape))
    z, rf = pl.pallas_call(
        functools.partial(_fused_body, tb=tb),
        grid=(B // tb,),
        in_specs=[
            pl.BlockSpec((15, 15, tb, 75), lambda i: (0, 0, i, 0)),
            full((75, 32)), full((1, 32)), full((1, 32)),
            full((5, 5, 32, 64)), full((1, 64)), full((1, 64)),
            full((5, 5, 64, 128)), full((1, 128)), full((1, 128)),
            full((5, 5, 128, 101)), full((1, 101)), full((1, 101)),
        ],
        out_shape=(jax.ShapeDtypeStruct((B, 100), jnp.float32),
                   jax.ShapeDtypeStruct((B, 1), jnp.float32)),
        out_specs=(pl.BlockSpec((tb, 100), lambda i: (i, 0)),
                   pl.BlockSpec((tb, 1), lambda i: (i, 0))),
        compiler_params=pltpu.CompilerParams(
            dimension_semantics=("parallel",)),
    )(p1, w1m, params[0][1], params[0][2],
      params[1][0], params[1][1], params[1][2],
      params[2][0], params[2][1], params[2][2],
      params[3][0], params[3][1], params[3][2])
    return z, rf[:, 0]


# speedup vs baseline: 503.4132x; 14.0185x over previous
import functools

import jax
import jax.numpy as jnp
import numpy as np
from jax.experimental import pallas as pl
from jax.experimental.pallas import tpu as pltpu

_TB = 256


def _band_matrices(w, j_in, j_real, j_out, c_major_rows=False):
    k, _, cin, cout = w.shape
    e = np.zeros((5, j_in, j_out), np.float32)
    for kw in range(5):
        for jp in range(j_out):
            j = 2 * jp + kw - 1
            if 0 <= j < j_real:
                e[kw, j, jp] = 1.0
    if c_major_rows:
        t = jnp.einsum("wjp,hwco->hcjpo", jnp.asarray(e), w)
    else:
        t = jnp.einsum("wjp,hwco->hjcpo", jnp.asarray(e), w)
    return t.reshape(5, j_in * cin, j_out * cout).astype(jnp.bfloat16)


def _strip_conv(ph0, ph1, wb_ref, s_ref, t_ref, ho, relu):
    d0 = jnp.dot(ph1[:, 0:ho - 1], wb_ref[0],
                 preferred_element_type=jnp.float32)
    acc = jnp.pad(d0, ((0, 0), (1, 0), (0, 0)))
    for kh, (src, u0) in enumerate(((ph0, 0), (ph1, 0), (ph0, 1), (ph1, 1)),
                                   start=1):
        acc = acc + jnp.dot(src[:, u0:u0 + ho], wb_ref[kh],
                            preferred_element_type=jnp.float32)
    out = acc * s_ref[...] + t_ref[...]
    if relu:
        out = jnp.maximum(out, 0.0)
    return out.astype(jnp.bfloat16)


def _l1_body(x_ref, wb_ref, s_ref, t_ref, o_ref, *, tb):
    xf = x_ref[...].astype(jnp.bfloat16)
    pm0 = jnp.concatenate([xf[:, c, :, 0:32] for c in range(3)], axis=-1)
    pm1 = jnp.concatenate([xf[:, c, :, 32:64] for c in range(3)], axis=-1)
    y = _strip_conv(pm0, pm1, wb_ref, s_ref, t_ref, 15, True)
    o_ref[...] = jnp.pad(y, ((0, 0), (0, 1), (0, 0)))


def _l2_body(a_ref, wb_ref, s_ref, t_ref, o_ref, *, tb):
    a = a_ref[...]
    y = _strip_conv(a[:, :, 0:512], a[:, :, 512:1024],
                    wb_ref, s_ref, t_ref, 7, True)
    o_ref[...] = jnp.pad(y, ((0, 0), (0, 1), (0, 0)))


def _l34_body(a_ref, wb3_ref, s3_ref, t3_ref, wb4_ref, s4_ref, t4_ref,
              z_ref, rf_ref, *, tb):
    a = a_ref[...]
    y = _strip_conv(a[:, :, 0:512], a[:, :, 512:1024],
                    wb3_ref, s3_ref, t3_ref, 3, True)
    acc = None
    for h in range(3):
        d = jnp.dot(y[:, h, :], wb4_ref[h],
                    preferred_element_type=jnp.float32)
        acc = d if acc is None else acc + d
    yout = acc * s4_ref[...] + t4_ref[...]
    z_ref[...] = yout[:, :100]
    rf_ref[...] = 1.0 / (1.0 + jnp.exp(-yout[:, 100:101]))


def kernel(x, w0, b0, scale0, shift0, w1, b1, scale1, shift1,
           w2, b2, scale2, shift2, w3, b3, scale3, shift3):
    B = x.shape[0]
    tb = min(_TB, B)
    grid = (B // tb,)
    par = pltpu.CompilerParams(dimension_semantics=("parallel",))
    full = lambda shape: pl.BlockSpec(shape, lambda i: (0,) * len(shape))

    def affine(b, s, t, j_out, cout, npad):
        se = jnp.tile(s.reshape(1, cout), (1, j_out))
        te = jnp.tile((b * s + t).reshape(1, cout), (1, j_out))
        pad = npad - j_out * cout
        return (jnp.pad(se, ((0, 0), (0, pad))).astype(jnp.float32),
                jnp.pad(te, ((0, 0), (0, pad))).astype(jnp.float32))

    wb1 = jnp.pad(_band_matrices(w0, 32, 32, 15, c_major_rows=True),
                  ((0, 0), (0, 0), (0, 512 - 15 * 32)))
    wb2 = jnp.pad(_band_matrices(w1, 16, 15, 7), ((0, 0), (0, 0), (0, 64)))
    wb3 = jnp.pad(_band_matrices(w2, 8, 7, 3), ((0, 0), (0, 0), (0, 128)))
    wb4 = jnp.pad(w3[1:4, 1:4], ((0, 0), (0, 1), (0, 0), (0, 0)))
    wb4 = wb4.reshape(3, 512, 101).astype(jnp.bfloat16)

    s1, t1 = affine(b0, scale0, shift0, 15, 32, 512)
    s2, t2 = affine(b1, scale1, shift1, 7, 64, 512)
    s3, t3 = affine(b2, scale2, shift2, 3, 128, 512)
    s4 = scale3.reshape(1, 101).astype(jnp.float32)
    t4 = (b3 * scale3 + shift3).reshape(1, 101).astype(jnp.float32)

    x4 = x.reshape(B, 3, 16, 64)
    y1 = pl.pallas_call(
        functools.partial(_l1_body, tb=tb),
        grid=grid,
        in_specs=[pl.BlockSpec((tb, 3, 16, 64), lambda i: (i, 0, 0, 0)),
                  full((5, 96, 512)), full((1, 512)), full((1, 512))],
        out_shape=jax.ShapeDtypeStruct((B, 16, 512), jnp.bfloat16),
        out_specs=pl.BlockSpec((tb, 16, 512), lambda i: (i, 0, 0)),
        compiler_params=par,
    )(x4, wb1, s1, t1)

    y2 = pl.pallas_call(
        functools.partial(_l2_body, tb=tb),
        grid=grid,
        in_specs=[pl.BlockSpec((tb, 8, 1024), lambda i: (i, 0, 0)),
                  full((5, 512, 512)), full((1, 512)), full((1, 512))],
        out_shape=jax.ShapeDtypeStruct((B, 8, 512), jnp.bfloat16),
        out_specs=pl.BlockSpec((tb, 8, 512), lambda i: (i, 0, 0)),
        compiler_params=par,
    )(y1.reshape(B, 8, 1024), wb2, s2, t2)

    z, rf = pl.pallas_call(
        functools.partial(_l34_body, tb=tb),
        grid=grid,
        in_specs=[pl.BlockSpec((tb, 4, 1024), lambda i: (i, 0, 0)),
                  full((5, 512, 512)), full((1, 512)), full((1, 512)),
                  full((3, 512, 101)), full((1, 101)), full((1, 101))],
        out_shape=(jax.ShapeDtypeStruct((B, 100), jnp.float32),
                   jax.ShapeDtypeStruct((B, 1), jnp.float32)),
        out_specs=(pl.BlockSpec((tb, 100), lambda i: (i, 0)),
                   pl.BlockSpec((tb, 1), lambda i: (i, 0))),
        compiler_params=par,
    )(y2.reshape(B, 4, 1024), wb3, s3, t3, wb4, s4, t4)
    return z, rf[:, 0]


# speedup vs baseline: 1526.3793x; 3.0321x over previous
import functools

import jax
import jax.numpy as jnp
import numpy as np
from jax.experimental import pallas as pl
from jax.experimental.pallas import tpu as pltpu

_TB = 256


def _band_matrices(w, j_in, j_real, j_out):
    k, _, cin, cout = w.shape
    e = np.zeros((5, j_in, j_out), np.float32)
    for kw in range(5):
        for jp in range(j_out):
            j = 2 * jp + kw - 1
            if 0 <= j < j_real:
                e[kw, j, jp] = 1.0
    t = jnp.einsum("wjp,hwco->hjcpo", jnp.asarray(e), w)
    return t.reshape(5, j_in * cin, j_out * cout).astype(jnp.bfloat16)


def _l1_body(x_ref, w_ref, s_ref, t_ref, o_ref, *, tb):
    f = x_ref[...].astype(jnp.bfloat16)
    zpad = jnp.zeros((tb, 512), jnp.bfloat16)
    cols = [zpad]
    for i in range(15):
        acc = None
        for c in range(3):
            if i == 0:
                lhs = f[:, c * 1024:c * 1024 + 128]
                wm = w_ref[c, 32:160]
            else:
                st = c * 1024 + (2 * i - 1) * 32
                lhs = f[:, st:st + 160]
                wm = w_ref[c]
            d = jnp.dot(lhs, wm, preferred_element_type=jnp.float32)
            acc = d if acc is None else acc + d
        y = acc * s_ref[...] + t_ref[...]
        cols.append(jnp.maximum(y, 0.0).astype(jnp.bfloat16))
    cols.append(zpad)
    o_ref[...] = jnp.concatenate(cols, axis=-1)


def _strip_body(a_ref, w_ref, s_ref, t_ref, o_ref, *, tb, ho):
    a = a_ref[...]
    zpad = jnp.zeros((tb, 512), jnp.bfloat16)
    cols = [zpad]
    for i in range(ho):
        d = jnp.dot(a[:, 1024 * i:1024 * i + 2560], w_ref[...],
                    preferred_element_type=jnp.float32)
        y = d * s_ref[...] + t_ref[...]
        cols.append(jnp.maximum(y, 0.0).astype(jnp.bfloat16))
    cols.append(zpad)
    o_ref[...] = jnp.concatenate(cols, axis=-1)


def _l34_body(a_ref, w3_ref, s3_ref, t3_ref, w4_ref, s4_ref, t4_ref,
              z_ref, rf_ref, *, tb):
    a = a_ref[...]
    zpad = jnp.zeros((tb, 512), jnp.bfloat16)
    cols = [zpad]
    for i in range(3):
        d = jnp.dot(a[:, 1024 * i:1024 * i + 2560], w3_ref[...],
                    preferred_element_type=jnp.float32)
        y = d * s3_ref[...] + t3_ref[...]
        cols.append(jnp.maximum(y, 0.0).astype(jnp.bfloat16))
    cols.append(zpad)
    lhs4 = jnp.concatenate(cols, axis=-1)
    acc = jnp.dot(lhs4, w4_ref[...], preferred_element_type=jnp.float32)
    yout = acc * s4_ref[...] + t4_ref[...]
    z_ref[...] = yout[:, :100]
    rf_ref[...] = 1.0 / (1.0 + jnp.exp(-yout[:, 100:101]))


def kernel(x, w0, b0, scale0, shift0, w1, b1, scale1, shift1,
           w2, b2, scale2, shift2, w3, b3, scale3, shift3):
    B = x.shape[0]
    tb = min(_TB, B)
    grid = (B // tb,)
    par = pltpu.CompilerParams(dimension_semantics=("parallel",))
    full = lambda shape: pl.BlockSpec(shape, lambda i: (0,) * len(shape))

    def affine(b, s, t, j_out, cout):
        se = jnp.tile(s.reshape(1, cout), (1, j_out))
        te = jnp.tile((b * s + t).reshape(1, cout), (1, j_out))
        pad = 512 - j_out * cout
        return (jnp.pad(se, ((0, 0), (0, pad))).astype(jnp.float32),
                jnp.pad(te, ((0, 0), (0, pad))).astype(jnp.float32))

    e1 = np.zeros((5, 32, 15), np.float32)
    for kw in range(5):
        for jp in range(15):
            j = 2 * jp + kw - 1
            if 0 <= j < 32:
                e1[kw, j, jp] = 1.0
    t1m = jnp.einsum("kjp,hkco->chjpo", jnp.asarray(e1), w0)
    wh1 = jnp.pad(t1m.reshape(3, 160, 480),
                  ((0, 0), (0, 0), (0, 32))).astype(jnp.bfloat16)
    wh2 = jnp.pad(_band_matrices(w1, 16, 15, 7),
                  ((0, 0), (0, 0), (0, 64))).reshape(2560, 512)
    wh3 = jnp.pad(_band_matrices(w2, 8, 7, 3),
                  ((0, 0), (0, 0), (0, 128))).reshape(2560, 512)
    wh4 = jnp.pad(w3[:, 1:4], ((0, 0), (0, 1), (0, 0), (0, 0)))
    wh4 = wh4.reshape(2560, 101).astype(jnp.bfloat16)

    s1, t1 = affine(b0, scale0, shift0, 15, 32)
    s2, t2 = affine(b1, scale1, shift1, 7, 64)
    s3, t3 = affine(b2, scale2, shift2, 3, 128)
    s4 = scale3.reshape(1, 101).astype(jnp.float32)
    t4 = (b3 * scale3 + shift3).reshape(1, 101).astype(jnp.float32)

    y1 = pl.pallas_call(
        functools.partial(_l1_body, tb=tb),
        grid=grid,
        in_specs=[pl.BlockSpec((tb, 3072), lambda i: (i, 0)),
                  full((3, 160, 512)), full((1, 512)), full((1, 512))],
        out_shape=jax.ShapeDtypeStruct((B, 17 * 512), jnp.bfloat16),
        out_specs=pl.BlockSpec((tb, 17 * 512), lambda i: (i, 0)),
        compiler_params=par,
    )(x.reshape(B, 3072), wh1, s1, t1)

    y2 = pl.pallas_call(
        functools.partial(_strip_body, tb=tb, ho=7),
        grid=grid,
        in_specs=[pl.BlockSpec((tb, 17 * 512), lambda i: (i, 0)),
                  full((2560, 512)), full((1, 512)), full((1, 512))],
        out_shape=jax.ShapeDtypeStruct((B, 9 * 512), jnp.bfloat16),
        out_specs=pl.BlockSpec((tb, 9 * 512), lambda i: (i, 0)),
        compiler_params=par,
    )(y1, wh2, s2, t2)

    z, rf = pl.pallas_call(
        functools.partial(_l34_body, tb=tb),
        grid=grid,
        in_specs=[pl.BlockSpec((tb, 9 * 512), lambda i: (i, 0)),
                  full((2560, 512)), full((1, 512)), full((1, 512)),
                  full((2560, 101)), full((1, 101)), full((1, 101))],
        out_shape=(jax.ShapeDtypeStruct((B, 100), jnp.float32),
                   jax.ShapeDtypeStruct((B, 1), jnp.float32)),
        out_specs=(pl.BlockSpec((tb, 100), lambda i: (i, 0)),
                   pl.BlockSpec((tb, 1), lambda i: (i, 0))),
        compiler_params=par,
    )(y2, wh3, s3, t3, wh4, s4, t4)
    return z, rf[:, 0]
